# baseline (speedup 1.0000x reference)
import functools

import jax
import jax.numpy as jnp
from jax import lax
from jax.experimental import pallas as pl
from jax.experimental.pallas import tpu as pltpu
from jax.experimental.pallas import tpu_sc as plsc

_EPS = 1e-5
_NC, _NS, _L = 2, 16, 16
_NW = _NC * _NS


def _rsqrt16(v):
    magic = jnp.full((_L,), 0x5F3759DF, jnp.int32)
    one = jnp.full((_L,), 1, jnp.int32)
    c15 = jnp.full((_L,), 1.5, jnp.float32)
    c05 = jnp.full((_L,), 0.5, jnp.float32)
    i = plsc.bitcast(v, jnp.int32)
    i = magic - lax.shift_right_arithmetic(i, one)
    r = plsc.bitcast(i, jnp.float32)
    half = v * c05
    for _ in range(4):
        r = r * (c15 - half * r * r)
    return r


def sc_fused(n, h, chunk=16):
    rows_per_w = n // _NW
    n_chunks = rows_per_w // chunk
    mesh = plsc.VectorSubcoreMesh(core_axis_name="c", subcore_axis_name="s")

    @functools.partial(
        pl.kernel, mesh=mesh,
        compiler_params=pltpu.CompilerParams(needs_layout_passes=False),
        out_type=jax.ShapeDtypeStruct((n, h), jnp.float32),
        scratch_types=[
            pltpu.VMEM((chunk,), jnp.int32),
            pltpu.VMEM((chunk, h), jnp.float32),
            pltpu.VMEM((chunk, h), jnp.float32),
            pltpu.VMEM((chunk, h), jnp.float32),
            pltpu.VMEM((h,), jnp.float32),
            pltpu.VMEM((h,), jnp.float32),
            pltpu.SemaphoreType.DMA,
        ],
    )
    def k(x_hbm, ids_hbm, tbl_hbm, g_hbm, b_hbm, out_hbm,
          idx_v, x_v, e_v, y_v, g_v, b_v, sem):
        wid = lax.axis_index("s") * _NC + lax.axis_index("c")
        pltpu.sync_copy(g_hbm, g_v)
        pltpu.sync_copy(b_hbm, b_v)
        rid = lax.broadcasted_iota(jnp.int32, (_L,), 0)
        inv_h = jnp.full((_L,), 1.0 / h, jnp.float32)
        eps = jnp.full((_L,), _EPS, jnp.float32)
        zero = jnp.zeros((_L,), jnp.float32)
        col0 = jnp.zeros((_L,), jnp.int32)
        onei = jnp.full((_L,), 1, jnp.int32)

        def chunk_body(c, carry):
            base = pl.multiple_of(wid * rows_per_w + c * chunk, chunk)
            pltpu.sync_copy(ids_hbm.at[pl.ds(base, chunk)], idx_v)
            pltpu.sync_copy(x_hbm.at[pl.ds(base, chunk)], x_v)
            pltpu.async_copy(tbl_hbm.at[idx_v], e_v, sem).wait()

            def pass1(j, st):
                s, q, col = st
                xv = plsc.load_gather(x_v, [rid, col])
                ev = plsc.load_gather(e_v, [rid, col])
                y = xv + ev
                plsc.store_scatter(y_v, [rid, col], y)
                return s + y, q + y * y, col + onei

            s, q, _ = lax.fori_loop(0, h, pass1, (zero, zero, col0))
            mean = s * inv_h
            var = q * inv_h - mean * mean
            rstd = _rsqrt16(var + eps)

            def pass2(j, st):
                (col,) = st
                yv = plsc.load_gather(y_v, [rid, col])
                gv = plsc.load_gather(g_v, [col])
                bv = plsc.load_gather(b_v, [col])
                out = (yv - mean) * rstd * gv + bv
                plsc.store_scatter(y_v, [rid, col], out)
                return (col + onei,)

            lax.fori_loop(0, h, pass2, (col0,))
            pltpu.sync_copy(y_v, out_hbm.at[pl.ds(base, chunk)])
            return carry

        lax.fori_loop(0, n_chunks, chunk_body, 0)

    return k


def kernel(batch_mention_emb, mention_type_ids, emb_table, ln_gamma, ln_beta):
    b, s, h = batch_mention_emb.shape
    n = b * s
    x = batch_mention_emb.reshape(n, h)
    ids = mention_type_ids.reshape(n).astype(jnp.int32)
    out = sc_fused(n, h)(x, ids, emb_table, ln_gamma, ln_beta)
    return out.reshape(b, s, h)


# speedup vs baseline: 36.6768x; 36.6768x over previous
import jax
import jax.numpy as jnp
from jax.experimental import pallas as pl
from jax.experimental.pallas import tpu as pltpu

_EPS = 1e-5


def _fused_body(ids_ref, x_ref, tbl_ref, g_ref, b_ref, o_ref):
    ids = ids_ref[0, 0, :]
    r = ids.shape[0]
    k = tbl_ref.shape[0]
    onehot = (ids[:, None] == jax.lax.broadcasted_iota(jnp.int32, (r, k), 1))
    e = jnp.dot(onehot.astype(jnp.float32), tbl_ref[...],
                preferred_element_type=jnp.float32)
    y = x_ref[...] + e
    mean = jnp.mean(y, axis=1, keepdims=True)
    yc = y - mean
    var = jnp.mean(yc * yc, axis=1, keepdims=True)
    o_ref[...] = yc * jax.lax.rsqrt(var + _EPS) * g_ref[...] + b_ref[...]


def kernel(batch_mention_emb, mention_type_ids, emb_table, ln_gamma, ln_beta):
    b, s, h = batch_mention_emb.shape
    n = b * s
    r = 2048
    nblk = n // r
    x = batch_mention_emb.reshape(n, h)
    ids = mention_type_ids.reshape(nblk, 1, r).astype(jnp.int32)
    k = 128
    tbl = jnp.zeros((k, h), emb_table.dtype).at[: emb_table.shape[0]].set(emb_table)
    out = pl.pallas_call(
        _fused_body,
        grid=(nblk,),
        in_specs=[
            pl.BlockSpec((1, 1, r), lambda i: (i, 0, 0)),
            pl.BlockSpec((r, h), lambda i: (i, 0)),
            pl.BlockSpec((k, h), lambda i: (0, 0)),
            pl.BlockSpec((1, h), lambda i: (0, 0)),
            pl.BlockSpec((1, h), lambda i: (0, 0)),
        ],
        out_specs=pl.BlockSpec((r, h), lambda i: (i, 0)),
        out_shape=jax.ShapeDtypeStruct((n, h), jnp.float32),
        compiler_params=pltpu.CompilerParams(dimension_semantics=("parallel",)),
    )(ids, x, tbl, ln_gamma.reshape(1, h), ln_beta.reshape(1, h))
    return out.reshape(b, s, h)
